# baseline (speedup 1.0000x reference)
import jax
import jax.numpy as jnp
from jax.experimental import pallas as pl
from jax.experimental.pallas import tpu as pltpu

_HIDDEN = 2048
_E = 64
_K = 8
_G = 8
_PG = _E // _G
_KG = 4
_SCALE = 2.5
_BT = 2048


def _router_block(w_ref, b_ref, h_ref, idx_ref, wt_ref):
    bt = h_ref.shape[0]
    logits = jax.lax.dot_general(
        w_ref[...], h_ref[...],
        dimension_numbers=(((1,), (1,)), ((), ())),
        preferred_element_type=jnp.float32)
    scores = jax.nn.sigmoid(logits)
    sfc = scores + b_ref[...]

    eiota = jax.lax.broadcasted_iota(jnp.int32, (_E, bt), 0)
    neg = jnp.float32(-1e30)

    g3 = sfc.reshape(_G, _PG, bt)
    pair = g3 + pltpu.roll(g3, _PG - 1, 1)
    for r in range(2, _PG // 2 + 1):
        pair = jnp.maximum(pair, g3 + pltpu.roll(g3, _PG - r, 1))
    gs = jnp.max(pair, axis=1)

    giota2 = jax.lax.broadcasted_iota(jnp.int32, (_G, bt), 0)
    rank = jnp.zeros((_G, bt), jnp.int32)
    one = jnp.ones((_G, bt), jnp.int32)
    zero = jnp.zeros((_G, bt), jnp.int32)
    for r in range(1, _G):
        rr = pltpu.roll(gs, _G - r, 0)
        beats = (rr > gs) | ((rr == gs) & (giota2 >= _G - r))
        rank = rank + jnp.where(beats, one, zero)
    sel = jnp.broadcast_to(
        (rank < _KG).reshape(_G, 1, bt), (_G, _PG, bt)).reshape(_E, bt)

    ms = jnp.where(sel, sfc, 0.0)

    kiota = jax.lax.broadcasted_iota(jnp.int32, (_K, bt), 0)
    idx_out = jnp.zeros((_K, bt), jnp.int32)
    w_out = jnp.zeros((_K, bt), jnp.float32)
    for j in range(_K):
        m = jnp.max(ms, axis=0, keepdims=True)
        fe = jnp.min(jnp.where(ms == m, eiota, _E), axis=0, keepdims=True)
        hit = eiota == fe
        wsel = jnp.max(jnp.where(hit, scores, neg), axis=0, keepdims=True)
        idx_out = jnp.where(kiota == j, fe, idx_out)
        w_out = jnp.where(kiota == j, wsel, w_out)
        ms = jnp.where(hit, neg, ms)

    denom = jnp.sum(w_out, axis=0, keepdims=True) + 1e-20
    wt_ref[...] = (w_out / denom * _SCALE).T
    idx_ref[...] = idx_out.T


def kernel(hidden_states, weight, e_score_correction_bias):
    tokens = hidden_states.shape[0]
    h = hidden_states.reshape(tokens, _HIDDEN).astype(jnp.float32)
    w = weight.astype(jnp.float32)
    b = e_score_correction_bias.astype(jnp.float32).reshape(_E, 1)
    bt = min(_BT, tokens)
    grid = tokens // bt

    idx_t, wts_t = pl.pallas_call(
        _router_block,
        grid=(grid,),
        in_specs=[
            pl.BlockSpec((_E, _HIDDEN), lambda i: (0, 0)),
            pl.BlockSpec((_E, 1), lambda i: (0, 0)),
            pl.BlockSpec((bt, _HIDDEN), lambda i: (i, 0)),
        ],
        out_specs=[
            pl.BlockSpec((bt, _K), lambda i: (i, 0)),
            pl.BlockSpec((bt, _K), lambda i: (i, 0)),
        ],
        out_shape=[
            jax.ShapeDtypeStruct((tokens, _K), jnp.int32),
            jax.ShapeDtypeStruct((tokens, _K), jnp.float32),
        ],
        compiler_params=pltpu.CompilerParams(
            dimension_semantics=("arbitrary",)),
    )(w, b, h)
    return idx_t, wts_t


# speedup vs baseline: 1.3513x; 1.3513x over previous
import jax
import jax.numpy as jnp
from jax.experimental import pallas as pl
from jax.experimental.pallas import tpu as pltpu

_HIDDEN = 2048
_E = 64
_K = 8
_G = 8
_PG = _E // _G
_KG = 4
_SCALE = 2.5
_BT = 2048


def _router_block(w_ref, b_ref, h_ref, idx_ref, wt_ref):
    bt = h_ref.shape[0]
    logits = jax.lax.dot_general(
        w_ref[...], h_ref[...],
        dimension_numbers=(((1,), (1,)), ((), ())),
        preferred_element_type=jnp.float32)
    scores = jax.nn.sigmoid(logits)
    sfc = scores + b_ref[...]

    eiota = jax.lax.broadcasted_iota(jnp.int32, (_E, bt), 0)
    neg = jnp.float32(-1e30)

    g3 = sfc.reshape(_G, _PG, bt)
    pair = g3 + pltpu.roll(g3, _PG - 1, 1)
    for r in range(2, _PG // 2 + 1):
        pair = jnp.maximum(pair, g3 + pltpu.roll(g3, _PG - r, 1))
    gs = jnp.max(pair, axis=1)

    giota2 = jax.lax.broadcasted_iota(jnp.int32, (_G, bt), 0)
    rank = jnp.zeros((_G, bt), jnp.int32)
    one = jnp.ones((_G, bt), jnp.int32)
    zero = jnp.zeros((_G, bt), jnp.int32)
    for r in range(1, _G):
        rr = pltpu.roll(gs, _G - r, 0)
        beats = (rr > gs) | ((rr == gs) & (giota2 >= _G - r))
        rank = rank + jnp.where(beats, one, zero)
    sel = jnp.broadcast_to(
        (rank < _KG).reshape(_G, 1, bt), (_G, _PG, bt)).reshape(_E, bt)

    ms = jnp.where(sel, sfc, 0.0)

    kiota = jax.lax.broadcasted_iota(jnp.int32, (_K, bt), 0)
    idx_out = jnp.zeros((_K, bt), jnp.int32)
    w_out = jnp.zeros((_K, bt), jnp.float32)
    for j in range(_K):
        m = jnp.max(ms, axis=0, keepdims=True)
        fe = jnp.min(jnp.where(ms == m, eiota, _E), axis=0, keepdims=True)
        hit = eiota == fe
        wsel = jnp.max(jnp.where(hit, scores, neg), axis=0, keepdims=True)
        idx_out = jnp.where(kiota == j, fe, idx_out)
        w_out = jnp.where(kiota == j, wsel, w_out)
        ms = jnp.where(hit, neg, ms)

    denom = jnp.sum(w_out, axis=0, keepdims=True) + 1e-20
    wt_ref[...] = w_out / denom * _SCALE
    idx_ref[...] = idx_out


def kernel(hidden_states, weight, e_score_correction_bias):
    tokens = hidden_states.shape[0]
    h = hidden_states.reshape(tokens, _HIDDEN).astype(jnp.float32)
    w = weight.astype(jnp.float32)
    b = e_score_correction_bias.astype(jnp.float32).reshape(_E, 1)
    bt = min(_BT, tokens)
    grid = tokens // bt

    idx_t, wts_t = pl.pallas_call(
        _router_block,
        grid=(grid,),
        in_specs=[
            pl.BlockSpec((_E, _HIDDEN), lambda i: (0, 0)),
            pl.BlockSpec((_E, 1), lambda i: (0, 0)),
            pl.BlockSpec((bt, _HIDDEN), lambda i: (i, 0)),
        ],
        out_specs=[
            pl.BlockSpec((_K, bt), lambda i: (0, i)),
            pl.BlockSpec((_K, bt), lambda i: (0, i)),
        ],
        out_shape=[
            jax.ShapeDtypeStruct((_K, tokens), jnp.int32),
            jax.ShapeDtypeStruct((_K, tokens), jnp.float32),
        ],
        compiler_params=pltpu.CompilerParams(
            dimension_semantics=("arbitrary",)),
    )(w, b, h)
    return idx_t.T, wts_t.T
